# baseline (speedup 1.0000x reference)
import functools

import jax
import jax.numpy as jnp
from jax import lax
from jax.experimental import pallas as pl
from jax.experimental.pallas import tpu as pltpu
from jax.experimental.pallas import tpu_sc as plsc

N_POSITION = 8192
D_MODEL = 1024
BATCH = 4
SEQ = 8192

NC, NS = 2, 16
NW = NC * NS
B_TOTAL = BATCH * SEQ
BPW = B_TOTAL // NW
R = 64
NCHUNK = BPW // R


@jax.jit
def _sc_gather(x_r, table):
    mesh = plsc.VectorSubcoreMesh(core_axis_name="c", subcore_axis_name="s")

    @functools.partial(
        pl.kernel,
        mesh=mesh,
        out_type=jax.ShapeDtypeStruct((B_TOTAL, D_MODEL), jnp.float32),
        scratch_types=[
            pltpu.VMEM((NCHUNK, R), jnp.int32),
            pltpu.VMEM((R, D_MODEL), jnp.float32),
            pltpu.SemaphoreType.DMA,
        ],
    )
    def k(x_hbm, table_hbm, out_hbm, idx_v, rows_v, sem):
        wid = lax.axis_index("s") * NC + lax.axis_index("c")
        base = wid * BPW
        pltpu.sync_copy(x_hbm.at[wid], idx_v)

        def body(c, carry):
            pltpu.async_copy(table_hbm.at[idx_v.at[c]], rows_v, sem).wait()
            pltpu.sync_copy(rows_v, out_hbm.at[pl.ds(base + c * R, R)])
            return carry

        lax.fori_loop(0, NCHUNK, body, 0)

    return k(x_r, table)


def kernel(x, table):
    x_r = x.reshape(NW, NCHUNK, R)
    out = _sc_gather(x_r, table)
    return out.reshape(BATCH, SEQ, D_MODEL)
